# baseline (speedup 1.0000x reference)
import jax
import jax.numpy as jnp
import numpy as np
from jax import lax
from jax.experimental import pallas as pl
from jax.experimental.pallas import tpu as pltpu
from jax.experimental.pallas import tpu_sc as plsc

_NUM_CLASSES = 1000
_SMOOTH = 0.1
_B = 16384
_L = 50

_HIT = np.float32(np.float32(1.0) * np.float32(1.0 - _SMOOTH)
                  + np.float32(_SMOOTH / _NUM_CLASSES))
_BG = np.float32(_SMOOTH / _NUM_CLASSES)

_NC = 2
_NS = 16
_NW = _NC * _NS
_BATCH_PER_W = _B // _NW
_BLK = 128
_NBLK = _BATCH_PER_W // _BLK
_SPLIT = 496
_PIECES = ((0, 16), (16, 16), (32, 16), (48, 2))


def _body(x_hbm, out_hbm, idx_v, idxt_v, bufa_v, bufb_v, sema, semb):
    cid = lax.axis_index("c")
    sid = lax.axis_index("s")
    wid = sid * _NC + cid

    hit = jnp.full((16,), _HIT, dtype=jnp.float32)
    bg = jnp.full((16,), _BG, dtype=jnp.float32)
    lanes = lax.iota(jnp.int32, 16)
    split = jnp.full((16,), _SPLIT, jnp.int32)
    b_locs = [jnp.full((16,), g * 16, jnp.int32) + lanes
              for g in range(_BLK // 16)]

    for buf, ncls in ((bufa_v, _SPLIT), (bufb_v, _NUM_CLASSES - _SPLIT)):
        def init_row(c, _, buf=buf):
            for g in range(_BLK // 16):
                buf[c, pl.ds(g * 16, 16)] = bg
            return 0
        lax.fori_loop(0, ncls, init_row, 0)

    def sweep(buf, bbase, value_vec, is_a):
        for (l0, nrows) in _PIECES:
            piece = idxt_v if nrows == 2 else idx_v
            pltpu.sync_copy(x_hbm.at[pl.ds(l0, nrows), pl.ds(bbase, _BLK)],
                            piece)

            def row_step(l, _, piece=piece):
                clss = [piece[l, pl.ds(g * 16, 16)]
                        for g in range(_BLK // 16)]
                for g in range(_BLK // 16):
                    if is_a:
                        plsc.store_scatter(buf, [clss[g], b_locs[g]],
                                           value_vec,
                                           mask=clss[g] < split)
                    else:
                        plsc.store_scatter(buf, [clss[g] - split, b_locs[g]],
                                           value_vec,
                                           mask=clss[g] >= split)
                return 0
            lax.fori_loop(0, nrows, row_step, 0)

    def phase(buf, sem, cls0, ncls, is_a, t, bbase):
        if t > 0:
            pltpu.make_async_copy(
                buf, out_hbm.at[pl.ds(cls0, ncls), pl.ds(bbase - _BLK, _BLK)],
                sem).wait()
            sweep(buf, bbase - _BLK, bg, is_a)
        sweep(buf, bbase, hit, is_a)
        pltpu.async_copy(
            buf, out_hbm.at[pl.ds(cls0, ncls), pl.ds(bbase, _BLK)], sem)

    for t in range(_NBLK):
        bbase = wid * _BATCH_PER_W + t * _BLK
        phase(bufa_v, sema, 0, _SPLIT, True, t, bbase)
        phase(bufb_v, semb, _SPLIT, _NUM_CLASSES - _SPLIT, False, t, bbase)

    last = wid * _BATCH_PER_W + (_NBLK - 1) * _BLK
    pltpu.make_async_copy(
        bufa_v, out_hbm.at[pl.ds(0, _SPLIT), pl.ds(last, _BLK)], sema).wait()
    pltpu.make_async_copy(
        bufb_v, out_hbm.at[pl.ds(_SPLIT, _NUM_CLASSES - _SPLIT),
                           pl.ds(last, _BLK)], semb).wait()


@jax.jit
def _multihot_t(x_t):
    mesh = plsc.VectorSubcoreMesh(core_axis_name="c", subcore_axis_name="s")
    fn = pl.kernel(
        _body,
        out_type=jax.ShapeDtypeStruct((_NUM_CLASSES, _B), jnp.float32),
        mesh=mesh,
        scratch_types=[
            pltpu.VMEM((16, _BLK), jnp.int32),
            pltpu.VMEM((2, _BLK), jnp.int32),
            pltpu.VMEM((_SPLIT, _BLK), jnp.float32),
            pltpu.VMEM((_NUM_CLASSES - _SPLIT, _BLK), jnp.float32),
            pltpu.SemaphoreType.DMA,
            pltpu.SemaphoreType.DMA,
        ],
        compiler_params=pltpu.CompilerParams(needs_layout_passes=False),
    )
    return fn(x_t)


def kernel(x):
    out_t = _multihot_t(x.astype(jnp.int32).T)
    return out_t.T


# speedup vs baseline: 1.3404x; 1.3404x over previous
import jax
import jax.numpy as jnp
import numpy as np
from jax import lax
from jax.experimental import pallas as pl
from jax.experimental.pallas import tpu as pltpu
from jax.experimental.pallas import tpu_sc as plsc

_NUM_CLASSES = 1000
_SMOOTH = 0.1
_B = 16384
_L = 50

_HIT = np.float32(np.float32(1.0) * np.float32(1.0 - _SMOOTH)
                  + np.float32(_SMOOTH / _NUM_CLASSES))
_BG = np.float32(_SMOOTH / _NUM_CLASSES)

_NC = 2
_NS = 16
_NW = _NC * _NS
_BATCH_PER_W = _B // _NW
_BLK = 128
_NBLK = _BATCH_PER_W // _BLK
_PIECES = ((0, 16), (16, 16), (32, 18))


def _body(x_hbm, out_hbm, idx_v, buf_v):
    cid = lax.axis_index("c")
    sid = lax.axis_index("s")
    wid = sid * _NC + cid

    hit = jnp.full((16,), _HIT, dtype=jnp.float32)
    bg = jnp.full((16,), _BG, dtype=jnp.float32)
    lanes = lax.iota(jnp.int32, 16)
    b_locs = [jnp.full((16,), g * 16, jnp.int32) + lanes
              for g in range(_BLK // 16)]

    def init_row(c, _):
        def init_col(g, __):
            buf_v[c, pl.ds(g * 16, 16)] = bg
            return 0
        lax.fori_loop(0, _BLK // 16, init_col, 0)
        return 0
    lax.fori_loop(0, _NUM_CLASSES, init_row, 0)

    def scatter_block(bbase, value_vec):
        for (l0, nrows) in _PIECES:
            piece = idx_v
            pltpu.sync_copy(x_hbm.at[pl.ds(l0, nrows), pl.ds(bbase, _BLK)],
                            piece.at[pl.ds(0, nrows), :])

            def row_step(l, _, piece=piece):
                clss = [piece[l, pl.ds(g * 16, 16)]
                        for g in range(_BLK // 16)]
                for g in range(_BLK // 16):
                    plsc.store_scatter(buf_v, [clss[g], b_locs[g]],
                                       value_vec)
                return 0
            lax.fori_loop(0, nrows, row_step, 0)

    for t in range(_NBLK):
        bbase = wid * _BATCH_PER_W + t * _BLK
        if t > 0:
            scatter_block(bbase - _BLK, bg)
        scatter_block(bbase, hit)
        pltpu.sync_copy(buf_v, out_hbm.at[:, pl.ds(bbase, _BLK)])


@jax.jit
def _multihot_t(x_t):
    mesh = plsc.VectorSubcoreMesh(core_axis_name="c", subcore_axis_name="s")
    fn = pl.kernel(
        _body,
        out_type=jax.ShapeDtypeStruct((_NUM_CLASSES, _B), jnp.float32),
        mesh=mesh,
        scratch_types=[
            pltpu.VMEM((18, _BLK), jnp.int32),
            pltpu.VMEM((_NUM_CLASSES, _BLK), jnp.float32),
        ],
        compiler_params=pltpu.CompilerParams(needs_layout_passes=False),
    )
    return fn(x_t)


def kernel(x):
    out_t = _multihot_t(x.astype(jnp.int32).T)
    return out_t.T


# speedup vs baseline: 1.3945x; 1.0404x over previous
import jax
import jax.numpy as jnp
import numpy as np
from jax import lax
from jax.experimental import pallas as pl
from jax.experimental.pallas import tpu as pltpu
from jax.experimental.pallas import tpu_sc as plsc

_NUM_CLASSES = 1000
_SMOOTH = 0.1
_B = 16384
_L = 50

_HIT = np.float32(np.float32(1.0) * np.float32(1.0 - _SMOOTH)
                  + np.float32(_SMOOTH / _NUM_CLASSES))
_BG = np.float32(_SMOOTH / _NUM_CLASSES)

_NC = 2
_NS = 16
_NW = _NC * _NS
_BATCH_PER_W = _B // _NW
_BLK = 128
_NBLK = _BATCH_PER_W // _BLK
_PIECES = ((0, 16), (16, 16), (32, 18))


def _body(x_hbm, out_hbm, idx_v, buf_v):
    cid = lax.axis_index("c")
    sid = lax.axis_index("s")
    wid = sid * _NC + cid

    hit = jnp.full((16,), _HIT, dtype=jnp.float32)
    bg = jnp.full((16,), _BG, dtype=jnp.float32)
    lanes = lax.iota(jnp.int32, 16)
    b_locs = [jnp.full((16,), g * 16, jnp.int32) + lanes
              for g in range(_BLK // 16)]

    def init_row(c, _):
        def init_col(g, __):
            buf_v[c, pl.ds(g * 16, 16)] = bg
            return 0
        lax.fori_loop(0, _BLK // 16, init_col, 0)
        return 0
    lax.fori_loop(0, _NUM_CLASSES, init_row, 0)

    def scan_piece(nrows, value_vec):
        def row_step(l, _):
            clss = [idx_v[l, pl.ds(g * 16, 16)]
                    for g in range(_BLK // 16)]
            for g in range(_BLK // 16):
                plsc.store_scatter(buf_v, [clss[g], b_locs[g]], value_vec)
            return 0
        lax.fori_loop(0, nrows, row_step, 0)

    def gather_piece(l0, nrows, bbase):
        pltpu.sync_copy(x_hbm.at[pl.ds(l0, nrows), pl.ds(bbase, _BLK)],
                        idx_v.at[pl.ds(0, nrows), :])

    for t in range(_NBLK):
        bbase = wid * _BATCH_PER_W + t * _BLK
        for (l0, nrows) in _PIECES:
            gather_piece(l0, nrows, bbase)
            scan_piece(nrows, hit)
        pltpu.sync_copy(buf_v, out_hbm.at[:, pl.ds(bbase, _BLK)])
        if t < _NBLK - 1:
            scan_piece(_PIECES[-1][1], bg)
            for (l0, nrows) in _PIECES[:-1]:
                gather_piece(l0, nrows, bbase)
                scan_piece(nrows, bg)


@jax.jit
def _multihot_t(x_t):
    mesh = plsc.VectorSubcoreMesh(core_axis_name="c", subcore_axis_name="s")
    fn = pl.kernel(
        _body,
        out_type=jax.ShapeDtypeStruct((_NUM_CLASSES, _B), jnp.float32),
        mesh=mesh,
        scratch_types=[
            pltpu.VMEM((18, _BLK), jnp.int32),
            pltpu.VMEM((_NUM_CLASSES, _BLK), jnp.float32),
        ],
        compiler_params=pltpu.CompilerParams(needs_layout_passes=False),
    )
    return fn(x_t)


def kernel(x):
    out_t = _multihot_t(x.astype(jnp.int32).T)
    return out_t.T
